# baseline (speedup 1.0000x reference)
import math

import jax
import jax.numpy as jnp
from jax.experimental import pallas as pl
from jax.experimental.pallas import tpu as pltpu

_SEQ = 200
_GD = 300
_D = 768
_BI = 8


def _body(tokens_ref, table_ref, w_ref, b_ref, out_ref, gath, vec, sem):
    i = pl.program_id(0)

    @pl.when(i == 0)
    def _():
        def start(t, c):
            pltpu.make_async_copy(
                table_ref.at[pl.ds(tokens_ref[t], 1), :],
                gath.at[pl.ds(t, 1), :],
                sem,
            ).start()
            return c

        jax.lax.fori_loop(0, _SEQ, start, 0)

        def wait(t, c):
            pltpu.make_async_copy(
                table_ref.at[pl.ds(0, 1), :],
                gath.at[pl.ds(t, 1), :],
                sem,
            ).wait()
            return c

        jax.lax.fori_loop(0, _SEQ, wait, 0)

        vec[...] = (
            jnp.dot(gath[...], w_ref[...], preferred_element_type=jnp.float32)
            + b_ref[...]
        )

    row = (i * _BI + jax.lax.broadcasted_iota(jnp.int32, (_BI, _D), 0)).astype(
        jnp.float32
    )
    col = jax.lax.broadcasted_iota(jnp.int32, (_BI, _D), 1)
    half = jnp.bitwise_and(col, -2).astype(jnp.float32)
    ang = row * jnp.exp(half * (-math.log(10000.0) / _D))
    pe = jnp.where(jnp.bitwise_and(col, 1) == 0, jnp.sin(ang), jnp.cos(ang))
    out_ref[...] = vec[...][None] + pe[:, None, :]


@jax.jit
def kernel(tokens, glove_table, W, b):
    b2 = b.reshape(1, _D)
    return pl.pallas_call(
        _body,
        grid=(_SEQ // _BI,),
        in_specs=[
            pl.BlockSpec(memory_space=pltpu.SMEM),
            pl.BlockSpec(memory_space=pl.ANY),
            pl.BlockSpec((_GD, _D), lambda i: (0, 0)),
            pl.BlockSpec((1, _D), lambda i: (0, 0)),
        ],
        out_specs=pl.BlockSpec((_BI, _SEQ, _D), lambda i: (i, 0, 0)),
        out_shape=jax.ShapeDtypeStruct((_SEQ, _SEQ, _D), jnp.float32),
        scratch_shapes=[
            pltpu.VMEM((_SEQ, _GD), jnp.float32),
            pltpu.VMEM((_SEQ, _D), jnp.float32),
            pltpu.SemaphoreType.DMA,
        ],
    )(tokens, glove_table, W, b2)
